# baseline (speedup 1.0000x reference)
import functools

import jax
import jax.numpy as jnp
from jax import lax
from jax.experimental import pallas as pl
from jax.experimental.pallas import tpu as pltpu
from jax.experimental.pallas import tpu_sc as plsc

_B, _H, _W, _K = 4, 512, 512, 8
_P = 200000
_AMBIENT = 0.3
_DIFFUSE = 0.7
_SPECULAR = 0.2

_CB = 7168
_PP = ((_P + _CB - 1) // _CB) * _CB

_NPIX = _B * _H * _W
_NW = 32
_CH = 1024
_SPLIT = 4
_PER_TILE = _NPIX // _NW
_NCHUNK = _PER_TILE // _CH


def _shade_body(aux_ref, pt_ref, ft_ref, nm_ref, ci_ref, out_ref):
    lx0, ly0, lz0 = aux_ref[4, 0], aux_ref[4, 1], aux_ref[4, 2]
    lin = 1.0 / jnp.maximum(jnp.sqrt(lx0 * lx0 + ly0 * ly0 + lz0 * lz0), 1e-12)
    lx, ly, lz = lx0 * lin, ly0 * lin, lz0 * lin

    ci = ci_ref[0:1, :]
    px, py, pz = pt_ref[0:1, :], pt_ref[1:2, :], pt_ref[2:3, :]
    nx, ny, nz = nm_ref[0:1, :], nm_ref[1:2, :], nm_ref[2:3, :]

    def pick(c):
        return jnp.where(
            ci == 0, aux_ref[0, c],
            jnp.where(ci == 1, aux_ref[1, c],
                      jnp.where(ci == 2, aux_ref[2, c], aux_ref[3, c])))

    vx = pick(0) - px
    vy = pick(1) - py
    vz = pick(2) - pz
    vinv = 1.0 / jnp.maximum(jnp.sqrt(vx * vx + vy * vy + vz * vz), 1e-12)
    vx, vy, vz = vx * vinv, vy * vinv, vz * vinv

    hx, hy, hz = lx + vx, ly + vy, lz + vz
    hinv = 1.0 / jnp.maximum(jnp.sqrt(hx * hx + hy * hy + hz * hz), 1e-12)
    hx, hy, hz = hx * hinv, hy * hinv, hz * hinv

    ndl = nx * lx + ny * ly + nz * lz
    base = _AMBIENT + _DIFFUSE * jnp.maximum(ndl, 0.0)

    ndh = jnp.maximum(nx * hx + ny * hy + nz * hz, 0.0)
    m2 = ndh * ndh
    m4 = m2 * m2
    m8 = m4 * m4
    m16 = m8 * m8
    spec = _SPECULAR * (m16 * m16)

    out_ref[...] = jnp.clip(ft_ref[...] * base + spec, 0.0, 1.0)


def _shade_table(points_t, features_t, normals_t, cloud_idx2, aux):
    grid = _PP // _CB
    return pl.pallas_call(
        _shade_body,
        grid=(grid,),
        in_specs=[
            pl.BlockSpec((8, 128), lambda i: (0, 0)),
            pl.BlockSpec((3, _CB), lambda i: (0, i)),
            pl.BlockSpec((3, _CB), lambda i: (0, i)),
            pl.BlockSpec((3, _CB), lambda i: (0, i)),
            pl.BlockSpec((1, _CB), lambda i: (0, i)),
        ],
        out_specs=pl.BlockSpec((3, _CB), lambda i: (0, i)),
        out_shape=jax.ShapeDtypeStruct((3, _PP), jnp.float32),
    )(aux, points_t, features_t, normals_t, cloud_idx2)


def _make_gather_kernel():
    mesh = plsc.VectorSubcoreMesh(core_axis_name="c", subcore_axis_name="s")

    @functools.partial(
        pl.kernel,
        out_type=jax.ShapeDtypeStruct((_NPIX, 4), jnp.float32),
        mesh=mesh,
        compiler_params=pltpu.CompilerParams(use_tc_tiling_on_sc=False),
        scratch_types=[
            pltpu.VMEM((2 * _CH,), jnp.int32),
            pltpu.VMEM((2 * _CH,), jnp.int32),
            pltpu.VMEM((2 * _CH, 4), jnp.float32),
            pltpu.VMEM((2 * _CH, 4), jnp.float32),
            pltpu.SemaphoreType.DMA,
            pltpu.SemaphoreType.DMA,
            pltpu.SemaphoreType.DMA,
            pltpu.SemaphoreType.DMA,
            pltpu.SemaphoreType.DMA,
            pltpu.SemaphoreType.DMA,
        ],
    )
    def gather_k(table_hbm, idx_hbm, out_hbm, i0, i1, r0, r1,
                 si0, si1, sg0, sg1, so0, so1):
        wid = lax.axis_index("s") * 2 + lax.axis_index("c")
        tb = wid * _PER_TILE
        ivs, rvs = (i0, i1), (r0, r1)
        sis, sgs, sos = (si0, si1), (sg0, sg1), (so0, so1)

        def idx_src(c):
            return idx_hbm.at[pl.ds(2 * (tb + c * _CH), 2 * _CH)]

        def out_dst(c):
            return out_hbm.at[pl.ds(tb + c * _CH, _CH)]

        sub = 2 * _CH // _SPLIT
        pix = sub // 2

        def gather_start(b):
            for k in range(_SPLIT):
                pltpu.async_copy(
                    table_hbm.at[ivs[b].at[pl.ds(k * sub, sub)]],
                    rvs[b].at[pl.ds(k * sub, sub)], sgs[b])

        def gather_wait(b):
            for k in range(_SPLIT):
                pltpu.make_async_copy(
                    table_hbm.at[ivs[b].at[pl.ds(k * sub, sub)]],
                    rvs[b].at[pl.ds(k * sub, sub)], sgs[b]).wait()

        def out_start(b, c):
            for k in range(_SPLIT):
                pltpu.async_copy(
                    rvs[b].at[pl.ds(k * sub, pix)],
                    out_hbm.at[pl.ds(tb + c * _CH + k * pix, pix)], sos[b])

        def out_wait(b, c):
            for k in range(_SPLIT):
                pltpu.make_async_copy(
                    rvs[b].at[pl.ds(k * sub, pix)],
                    out_hbm.at[pl.ds(tb + c * _CH + k * pix, pix)], sos[b]).wait()

        pltpu.async_copy(idx_src(0), i0, si0)
        pltpu.async_copy(idx_src(1), i1, si1)
        pltpu.make_async_copy(idx_src(0), i0, si0).wait()
        gather_start(0)

        def body(step, carry):
            for b in (0, 1):
                c = 2 * step + b
                nb = 1 - b

                @pl.when(c + 1 < _NCHUNK)
                def _():
                    pltpu.make_async_copy(idx_src(c + 1), ivs[nb], sis[nb]).wait()

                    @pl.when(c >= 1)
                    def _():
                        out_wait(nb, c - 1)

                    gather_start(nb)

                gather_wait(b)

                @pl.when(c + 2 < _NCHUNK)
                def _():
                    pltpu.async_copy(idx_src(c + 2), ivs[b], sis[b])

                out_start(b, c)
            return carry

        lax.fori_loop(0, _NCHUNK // 2, body, 0)
        out_wait(0, _NCHUNK - 2)
        out_wait(1, _NCHUNK - 1)

    return gather_k


_gather_rows = _make_gather_kernel()


def kernel(idx, points, features, normals, cloud_idx, cam_centers, light_dir):
    pad = _PP - _P
    pts_t = jnp.pad(points, ((0, pad), (0, 0))).T
    ft_t = jnp.pad(features, ((0, pad), (0, 0))).T
    nm_t = jnp.pad(normals, ((0, pad), (0, 0))).T
    ci2 = jnp.pad(cloud_idx.astype(jnp.int32), (0, pad)).reshape(1, _PP)
    aux = jnp.zeros((8, 128), jnp.float32)
    aux = aux.at[:4, :3].set(cam_centers).at[4, :3].set(light_dir)

    shaded_planar = _shade_table(pts_t, ft_t, nm_t, ci2, aux)
    bg_row = jnp.ones((1, 4), jnp.float32)
    shaded4 = jnp.concatenate(
        [shaded_planar[:, :_P].T, jnp.zeros((_P, 1), jnp.float32)], axis=1)
    table = jnp.concatenate([bg_row, shaded4], axis=0)

    v = idx[..., 0].reshape(-1) + 1
    dv = jnp.stack([2 * v, 2 * v + 1], axis=1).reshape(-1)
    out = _gather_rows(table, dv)
    return out[:, :3].reshape(_B, _H, _W, 3)


# speedup vs baseline: 1.8533x; 1.8533x over previous
import functools

import jax
import jax.numpy as jnp
from jax import lax
from jax.experimental import pallas as pl
from jax.experimental.pallas import tpu as pltpu
from jax.experimental.pallas import tpu_sc as plsc

_B, _H, _W, _K = 4, 512, 512, 8
_P = 200000
_AMBIENT = 0.3
_DIFFUSE = 0.7
_SPECULAR = 0.2

_CB = 7168
_PP = ((_P + _CB - 1) // _CB) * _CB
_PTAB = _P + 1

_NPIX = _B * _H * _W
_NW = 32
_CH = 1024
_SPLIT = 4
_PER_TILE = _NPIX // _NW
_NCHUNK = _PER_TILE // _CH

def _shade_body(aux_ref, pt_ref, ft_ref, nm_ref, ci_ref, out_ref):
    lx0, ly0, lz0 = aux_ref[4, 0], aux_ref[4, 1], aux_ref[4, 2]
    lin = 1.0 / jnp.maximum(jnp.sqrt(lx0 * lx0 + ly0 * ly0 + lz0 * lz0), 1e-12)
    lx, ly, lz = lx0 * lin, ly0 * lin, lz0 * lin

    ci = ci_ref[0:1, :]
    px, py, pz = pt_ref[0:1, :], pt_ref[1:2, :], pt_ref[2:3, :]
    nx, ny, nz = nm_ref[0:1, :], nm_ref[1:2, :], nm_ref[2:3, :]

    def pick(c):
        return jnp.where(
            ci == 0, aux_ref[0, c],
            jnp.where(ci == 1, aux_ref[1, c],
                      jnp.where(ci == 2, aux_ref[2, c], aux_ref[3, c])))

    vx = pick(0) - px
    vy = pick(1) - py
    vz = pick(2) - pz
    vinv = 1.0 / jnp.maximum(jnp.sqrt(vx * vx + vy * vy + vz * vz), 1e-12)
    vx, vy, vz = vx * vinv, vy * vinv, vz * vinv

    hx, hy, hz = lx + vx, ly + vy, lz + vz
    hinv = 1.0 / jnp.maximum(jnp.sqrt(hx * hx + hy * hy + hz * hz), 1e-12)
    hx, hy, hz = hx * hinv, hy * hinv, hz * hinv

    ndl = nx * lx + ny * ly + nz * lz
    base = _AMBIENT + _DIFFUSE * jnp.maximum(ndl, 0.0)

    ndh = jnp.maximum(nx * hx + ny * hy + nz * hz, 0.0)
    m2 = ndh * ndh
    m4 = m2 * m2
    m8 = m4 * m4
    m16 = m8 * m8
    spec = _SPECULAR * (m16 * m16)

    out_ref[...] = jnp.clip(ft_ref[...] * base + spec, 0.0, 1.0)


def _shade_table(points_t, features_t, normals_t, cloud_idx2, aux):
    grid = _PP // _CB
    return pl.pallas_call(
        _shade_body,
        grid=(grid,),
        in_specs=[
            pl.BlockSpec((8, 128), lambda i: (0, 0)),
            pl.BlockSpec((3, _CB), lambda i: (0, i)),
            pl.BlockSpec((3, _CB), lambda i: (0, i)),
            pl.BlockSpec((3, _CB), lambda i: (0, i)),
            pl.BlockSpec((1, _CB), lambda i: (0, i)),
        ],
        out_specs=pl.BlockSpec((3, _CB), lambda i: (0, i)),
        out_shape=jax.ShapeDtypeStruct((3, _PP), jnp.float32),
    )(aux, points_t, features_t, normals_t, cloud_idx2)


def _make_gather_kernel():
    mesh = plsc.VectorSubcoreMesh(core_axis_name="c", subcore_axis_name="s")

    @functools.partial(
        pl.kernel,
        out_type=jax.ShapeDtypeStruct((_NPIX, 2), jnp.float32),
        mesh=mesh,
        compiler_params=pltpu.CompilerParams(use_tc_tiling_on_sc=False),
        scratch_types=[
            pltpu.VMEM((_CH,), jnp.int32),
            pltpu.VMEM((_CH,), jnp.int32),
            pltpu.VMEM((_CH, 2), jnp.float32),
            pltpu.VMEM((_CH, 2), jnp.float32),
            pltpu.SemaphoreType.DMA,
            pltpu.SemaphoreType.DMA,
            pltpu.SemaphoreType.DMA,
            pltpu.SemaphoreType.DMA,
            pltpu.SemaphoreType.DMA,
            pltpu.SemaphoreType.DMA,
        ],
    )
    def gather_k(table_hbm, idx_hbm, out_hbm, i0, i1, r0, r1,
                 si0, si1, sg0, sg1, so0, so1):
        wid = lax.axis_index("s") * 2 + lax.axis_index("c")
        tb = wid * _PER_TILE
        ivs, rvs = (i0, i1), (r0, r1)
        sis, sgs, sos = (si0, si1), (sg0, sg1), (so0, so1)

        sub = _CH // _SPLIT

        def idx_src(c):
            return idx_hbm.at[pl.ds(tb + c * _CH, _CH)]

        def gather_start(b):
            for k in range(_SPLIT):
                pltpu.async_copy(
                    table_hbm.at[ivs[b].at[pl.ds(k * sub, sub)]],
                    rvs[b].at[pl.ds(k * sub, sub)], sgs[b])

        def gather_wait(b):
            for k in range(_SPLIT):
                pltpu.make_async_copy(
                    table_hbm.at[ivs[b].at[pl.ds(k * sub, sub)]],
                    rvs[b].at[pl.ds(k * sub, sub)], sgs[b]).wait()

        def out_start(b, c):
            pltpu.async_copy(rvs[b], out_hbm.at[pl.ds(tb + c * _CH, _CH)],
                             sos[b])

        def out_wait(b, c):
            pltpu.make_async_copy(
                rvs[b], out_hbm.at[pl.ds(tb + c * _CH, _CH)], sos[b]).wait()

        pltpu.async_copy(idx_src(0), i0, si0)
        pltpu.async_copy(idx_src(1), i1, si1)
        pltpu.make_async_copy(idx_src(0), i0, si0).wait()
        gather_start(0)

        def body(step, carry):
            for b in (0, 1):
                c = 2 * step + b
                nb = 1 - b

                @pl.when(c + 1 < _NCHUNK)
                def _():
                    pltpu.make_async_copy(idx_src(c + 1), ivs[nb], sis[nb]).wait()

                    @pl.when(c >= 1)
                    def _():
                        out_wait(nb, c - 1)

                    gather_start(nb)

                gather_wait(b)

                @pl.when(c + 2 < _NCHUNK)
                def _():
                    pltpu.async_copy(idx_src(c + 2), ivs[b], sis[b])

                out_start(b, c)
            return carry

        lax.fori_loop(0, _NCHUNK // 2, body, 0)
        out_wait(0, _NCHUNK - 2)
        out_wait(1, _NCHUNK - 1)

    return gather_k


_gather_rows = _make_gather_kernel()


def kernel(idx, points, features, normals, cloud_idx, cam_centers, light_dir):
    pad = _PP - _P
    pts_t = jnp.pad(points, ((0, pad), (0, 0))).T
    ft_t = jnp.pad(features, ((0, pad), (0, 0))).T
    nm_t = jnp.pad(normals, ((0, pad), (0, 0))).T
    ci2 = jnp.pad(cloud_idx.astype(jnp.int32), (0, pad)).reshape(1, _PP)
    aux = jnp.zeros((8, 128), jnp.float32)
    aux = aux.at[:4, :3].set(cam_centers).at[4, :3].set(light_dir)

    shaded = _shade_table(pts_t, ft_t, nm_t, ci2, aux)

    sh16 = shaded[:, :_P].astype(jnp.bfloat16)
    zero = jnp.zeros((1, _P), jnp.bfloat16)
    quad = jnp.stack([sh16[0], sh16[1], sh16[2], zero[0]], axis=1)
    bgq = jnp.ones((1, 4), jnp.bfloat16).at[0, 3].set(0)
    quad = jnp.concatenate([bgq, quad], axis=0)
    table = lax.bitcast_convert_type(
        quad.reshape(_PTAB, 2, 2), jnp.float32)

    idx0 = idx[..., 0].reshape(_NPIX)
    v = jnp.where(idx0 < 0, 0, idx0 + 1)
    out2 = _gather_rows(table, v)

    out4 = lax.bitcast_convert_type(out2, jnp.bfloat16).reshape(_NPIX, 4)
    return out4[:, :3].astype(jnp.float32).reshape(_B, _H, _W, 3)
